# baseline (speedup 1.0000x reference)
import functools

import jax
import jax.numpy as jnp
from jax import lax
from jax.experimental import pallas as pl
from jax.experimental.pallas import tpu as pltpu
from jax.experimental.pallas import tpu_sc as plsc

_DIM = 256
_N = 10000
_E = 160000
_HALF = 128
_NB = 10
_BLK = _N // _NB


def _gelu(x):
    return 0.5 * x * (1.0 + lax.erf(x * 0.7071067811865476))


def _lin1_body(hc_ref, hn_ref, cnt_ref, w1c_ref, b1c_ref, w1n_ref, b1n_ref,
               x4_ref):
    h1c = _gelu(jnp.dot(hc_ref[...], w1c_ref[...],
                        preferred_element_type=jnp.float32) + b1c_ref[...])
    h1n = _gelu(jnp.dot(hn_ref[...], w1n_ref[...],
                        preferred_element_type=jnp.float32) + b1n_ref[...])
    for r, h1 in ((0, h1c), (1, h1c), (2, h1n), (3, h1n)):
        s = lax.rsqrt(jnp.maximum(cnt_ref[:, r], 1.0))[:, None]
        xs = h1 * s
        x4_ref[r, 0] = xs[:, :_HALF]
        x4_ref[r, 1] = xs[:, _HALF:]


def _lin1_call(h_cell, h_net, cnt_src, W1_cell, b1_cell, W1_net, b1_net):
    return pl.pallas_call(
        _lin1_body,
        grid=(_NB,),
        in_specs=[
            pl.BlockSpec((_BLK, _DIM), lambda i: (i, 0)),
            pl.BlockSpec((_BLK, _DIM), lambda i: (i, 0)),
            pl.BlockSpec((_BLK, 4), lambda i: (i, 0)),
            pl.BlockSpec((_DIM, _DIM), lambda i: (0, 0)),
            pl.BlockSpec((1, _DIM), lambda i: (0, 0)),
            pl.BlockSpec((_DIM, _DIM), lambda i: (0, 0)),
            pl.BlockSpec((1, _DIM), lambda i: (0, 0)),
        ],
        out_specs=pl.BlockSpec((4, 2, _BLK, _HALF), lambda i: (0, 0, i, 0)),
        out_shape=jax.ShapeDtypeStruct((4, 2, _N, _HALF), jnp.float32),
    )(h_cell, h_net, cnt_src, W1_cell, b1_cell.reshape(1, _DIM),
      W1_net, b1_net.reshape(1, _DIM))


def _out_body(hc_ref, hn_ref, cnt_ref, agg_ref, w4_ref, bsum_ref,
              w2c_ref, b2c_ref, w2n_ref, b2n_ref, yc_ref, yn_ref):
    sin = [lax.rsqrt(jnp.maximum(cnt_ref[:, r], 1.0))[:, None]
           for r in range(4)]

    def conv(r_a, r_b, bias_row):
        acc = bsum_ref[bias_row, :][None, :]
        acc = jnp.broadcast_to(acc, (_BLK, 2 * _DIM)).astype(jnp.float32)
        for r in (r_a, r_b):
            for h in (0, 1):
                acc = acc + jnp.dot(agg_ref[r, h] * sin[r], w4_ref[r, h],
                                    preferred_element_type=jnp.float32)
        return acc

    h2c = _gelu(conv(0, 2, 0))
    h2n = _gelu(conv(1, 3, 1))
    out_c = _gelu(jnp.dot(h2c, w2c_ref[...],
                          preferred_element_type=jnp.float32) + b2c_ref[...])
    out_n = _gelu(jnp.dot(h2n, w2n_ref[...],
                          preferred_element_type=jnp.float32) + b2n_ref[...])
    yc_ref[...] = hc_ref[...] + out_c
    yn_ref[...] = hn_ref[...] + out_n


def _out_call(h_cell, h_net, cnt_dst, agg4, W4, bsum, W2_cell, b2_cell,
              W2_net, b2_net):
    return pl.pallas_call(
        _out_body,
        grid=(_NB,),
        in_specs=[
            pl.BlockSpec((_BLK, _DIM), lambda i: (i, 0)),
            pl.BlockSpec((_BLK, _DIM), lambda i: (i, 0)),
            pl.BlockSpec((_BLK, 4), lambda i: (i, 0)),
            pl.BlockSpec((4, 2, _BLK, _HALF), lambda i: (0, 0, i, 0)),
            pl.BlockSpec((4, 2, _HALF, 2 * _DIM), lambda i: (0, 0, 0, 0)),
            pl.BlockSpec((2, 2 * _DIM), lambda i: (0, 0)),
            pl.BlockSpec((2 * _DIM, _DIM), lambda i: (0, 0)),
            pl.BlockSpec((1, _DIM), lambda i: (0, 0)),
            pl.BlockSpec((2 * _DIM, _DIM), lambda i: (0, 0)),
            pl.BlockSpec((1, _DIM), lambda i: (0, 0)),
        ],
        out_specs=[
            pl.BlockSpec((_BLK, _DIM), lambda i: (i, 0)),
            pl.BlockSpec((_BLK, _DIM), lambda i: (i, 0)),
        ],
        out_shape=[
            jax.ShapeDtypeStruct((_N, _DIM), jnp.float32),
            jax.ShapeDtypeStruct((_N, _DIM), jnp.float32),
        ],
    )(h_cell, h_net, cnt_dst, agg4, W4, bsum,
      W2_cell, b2_cell.reshape(1, _DIM), W2_net, b2_net.reshape(1, _DIM))


_NC = 2
_NS = 16
_CH = 125
_NCHUNK = _E // (_NS * _CH)


def _hist_body(idx_ref, zeros_ref, ones_ref, out_ref, acc, idx_v, ones_v,
               zero_v, sem):
    c = lax.axis_index("c")
    s = lax.axis_index("s")
    pltpu.sync_copy(ones_ref, ones_v)

    pltpu.sync_copy(zeros_ref, zero_v)

    @pl.when(s < 10)
    def _():
        pltpu.sync_copy(zero_v, acc.at[pl.ds(s * 4000, 4000)])

    for k in range(4):
        pltpu.sync_copy(idx_ref.at[4 * c + k, s], idx_v.at[k])
    plsc.subcore_barrier()

    for k in range(4):
        def fire(j, carry, k=k):
            pltpu.async_copy(ones_v, acc.at[idx_v.at[k, j]], sem, add=True)
            return carry
        lax.fori_loop(0, _NCHUNK, fire, 0)

    def drain(j, carry):
        pltpu.make_async_copy(ones_v, acc.at[idx_v.at[0, 0]], sem).wait()
        return carry
    lax.fori_loop(0, 4 * _NCHUNK, drain, 0)
    plsc.subcore_barrier()

    @pl.when(s < 10)
    def _():
        for k in range(4):
            bounce = zero_v.at[pl.ds(0, 1000)]
            pltpu.sync_copy(acc.at[pl.ds(k * _N + s * 1000, 1000)], bounce)
            pltpu.sync_copy(
                bounce,
                out_ref.at[pl.ds((4 * c + k) * _N + s * 1000, 1000)])


def _histograms(idx8):
    mesh = plsc.VectorSubcoreMesh(core_axis_name="c", subcore_axis_name="s")
    hist = functools.partial(
        pl.kernel,
        out_type=jax.ShapeDtypeStruct((8 * _N,), jnp.float32),
        mesh=mesh,
        scratch_types=[
            pltpu.VMEM_SHARED((4 * _N,), jnp.float32),
            pltpu.VMEM((4, _NCHUNK, _CH), jnp.int32),
            pltpu.VMEM((_CH,), jnp.float32),
            pltpu.VMEM((4000,), jnp.float32),
            pltpu.SemaphoreType.DMA,
        ],
    )(_hist_body)
    idx_adj = idx8 + (jnp.arange(8, dtype=jnp.int32) % 4 * _N)[:, None]
    out = hist(idx_adj.reshape(8, _NS, _NCHUNK, _CH),
               jnp.zeros((4000,), jnp.float32),
               jnp.ones((_CH,), jnp.float32))
    return out.reshape(8, _N)


_NQ = 2
_QCH = _NCHUNK // _NQ
_MROWS = [(q * 104, 104) for q in range(9)] + [(936, 64)]


def _seg_body(x_ref, src_ref, dst_ref, zeros_ref, out_ref,
              acc, src_v, dst_v, bufa, bufb, gsem, ssem, gsem2, ssem2):
    c = lax.axis_index("c")
    s = lax.axis_index("s")

    def _gw(sem):
        pltpu.make_async_copy(x_ref.at[0, 0].at[src_v.at[0]], bufb,
                              sem).wait()

    def _sw(sem):
        pltpu.make_async_copy(bufb, acc.at[dst_v.at[0]], sem).wait()

    gwait = lambda: _gw(gsem)
    gwait2 = lambda: _gw(gsem2)
    swait = lambda: _sw(ssem)
    swait2 = lambda: _sw(ssem2)

    for r in range(4):
        pltpu.sync_copy(zeros_ref, bufa)

        @pl.when(s < 10)
        def _():
            for off, n in _MROWS:
                pltpu.async_copy(bufa.at[pl.ds(0, n)],
                                 acc.at[pl.ds(s * 1000 + off, n)], ssem)
            for off, n in _MROWS:
                pltpu.make_async_copy(
                    bufa.at[pl.ds(0, n)],
                    acc.at[pl.ds(s * 1000 + off, n)], ssem).wait()
        plsc.subcore_barrier()

        table = x_ref.at[r, c]
        for q4 in range(_NQ):
            pltpu.sync_copy(src_ref.at[r, s, q4], src_v)
            pltpu.sync_copy(dst_ref.at[r, s, q4], dst_v)
            pltpu.async_copy(table.at[src_v.at[0]], bufa, gsem)
            pltpu.async_copy(table.at[src_v.at[1]], bufb, gsem2)

            def pair(t, carry):
                gwait()
                pltpu.async_copy(bufa, acc.at[dst_v.at[2 * t]], ssem,
                                 add=True)
                gwait2()
                pltpu.async_copy(bufb, acc.at[dst_v.at[2 * t + 1]], ssem2,
                                 add=True)

                @pl.when(t < _QCH // 2 - 1)
                def _():
                    swait()
                    pltpu.async_copy(table.at[src_v.at[2 * t + 2]], bufa,
                                     gsem)
                    swait2()
                    pltpu.async_copy(table.at[src_v.at[2 * t + 3]], bufb,
                                     gsem2)
                return carry

            lax.fori_loop(0, _QCH // 2, pair, 0)
            swait()
            swait2()
        plsc.subcore_barrier()

        @pl.when(s < 10)
        def _():
            bufs = (bufa, bufb)
            for q, (off, n) in enumerate(_MROWS):
                buf = bufs[q % 2].at[pl.ds(0, n)]
                rows = pl.ds(s * 1000 + off, n)
                if q >= 2:
                    poff, pn = _MROWS[q - 2]
                    pbuf = bufs[q % 2].at[pl.ds(0, pn)]
                    prows = pl.ds(s * 1000 + poff, pn)
                    pltpu.make_async_copy(pbuf, out_ref.at[r, c].at[prows],
                                          gsem).wait()
                pltpu.async_copy(acc.at[rows], buf, ssem)
                pltpu.make_async_copy(acc.at[rows], buf, ssem).wait()
                pltpu.async_copy(buf, out_ref.at[r, c].at[rows], gsem)
            for q in (8, 9):
                off, n = _MROWS[q]
                buf = bufs[q % 2].at[pl.ds(0, n)]
                rows = pl.ds(s * 1000 + off, n)
                pltpu.make_async_copy(buf, out_ref.at[r, c].at[rows],
                                      gsem).wait()
        plsc.subcore_barrier()


def _segsums(x4, edges):
    mesh = plsc.VectorSubcoreMesh(core_axis_name="c", subcore_axis_name="s")
    seg = functools.partial(
        pl.kernel,
        out_type=jax.ShapeDtypeStruct((4, 2, _N, _HALF), jnp.float32),
        mesh=mesh,
        scratch_types=[
            pltpu.VMEM_SHARED((_N, _HALF), jnp.float32),
            pltpu.VMEM((_QCH, _CH), jnp.int32),
            pltpu.VMEM((_QCH, _CH), jnp.int32),
            pltpu.VMEM((_CH, _HALF), jnp.float32),
            pltpu.VMEM((_CH, _HALF), jnp.float32),
            pltpu.SemaphoreType.DMA,
            pltpu.SemaphoreType.DMA,
            pltpu.SemaphoreType.DMA,
            pltpu.SemaphoreType.DMA,
        ],
    )(_seg_body)
    src4 = jnp.stack([e[0] for e in edges]).reshape(4, _NS, _NQ, _QCH, _CH)
    dst4 = jnp.stack([e[1] for e in edges]).reshape(4, _NS, _NQ, _QCH, _CH)
    return seg(x4, src4, dst4, jnp.zeros((_CH, _HALF), jnp.float32))


def kernel(h_cell, h_net, edge_cc, edge_cn, edge_nc, edge_nn,
           W1_cell, b1_cell, W1_net, b1_net,
           Wcc, bcc, Wcn, bcn, Wnc, bnc, Wnn, bnn,
           W2_cell, b2_cell, W2_net, b2_net):
    edges = [edge_cc, edge_cn, edge_nc, edge_nn]
    idx8 = jnp.stack([e[0] for e in edges] + [e[1] for e in edges])

    counts = _histograms(idx8)
    cnt_src = counts[:4].T
    cnt_dst = counts[4:].T

    x4 = _lin1_call(h_cell, h_net, cnt_src, W1_cell, b1_cell, W1_net, b1_net)

    agg4 = _segsums(x4, edges)

    W4 = jnp.stack([jnp.stack([W[:_HALF], W[_HALF:]])
                    for W in (Wcc, Wcn, Wnc, Wnn)])
    bsum = jnp.stack([bcc + bnc, bcn + bnn])

    yc, yn = _out_call(h_cell, h_net, cnt_dst, agg4, W4, bsum,
                       W2_cell, b2_cell, W2_net, b2_net)
    return (yc, yn)


# speedup vs baseline: 1.0944x; 1.0944x over previous
import functools

import jax
import jax.numpy as jnp
from jax import lax
from jax.experimental import pallas as pl
from jax.experimental.pallas import tpu as pltpu
from jax.experimental.pallas import tpu_sc as plsc

_DIM = 256
_N = 10000
_E = 160000
_HALF = 128
_NB = 10
_BLK = _N // _NB


def _gelu(x):
    return 0.5 * x * (1.0 + lax.erf(x * 0.7071067811865476))


def _lin1_body(hc_ref, hn_ref, cnt_ref, w1c_ref, b1c_ref, w1n_ref, b1n_ref,
               x4_ref):
    h1c = _gelu(jnp.dot(hc_ref[...], w1c_ref[...],
                        preferred_element_type=jnp.float32) + b1c_ref[...])
    h1n = _gelu(jnp.dot(hn_ref[...], w1n_ref[...],
                        preferred_element_type=jnp.float32) + b1n_ref[...])
    for r, h1 in ((0, h1c), (1, h1c), (2, h1n), (3, h1n)):
        s = lax.rsqrt(jnp.maximum(cnt_ref[:, r], 1.0))[:, None]
        xs = h1 * s
        x4_ref[r, 0] = xs[:, :_HALF]
        x4_ref[r, 1] = xs[:, _HALF:]


def _lin1_call(h_cell, h_net, cnt_src, W1_cell, b1_cell, W1_net, b1_net):
    return pl.pallas_call(
        _lin1_body,
        grid=(_NB,),
        in_specs=[
            pl.BlockSpec((_BLK, _DIM), lambda i: (i, 0)),
            pl.BlockSpec((_BLK, _DIM), lambda i: (i, 0)),
            pl.BlockSpec((_BLK, 4), lambda i: (i, 0)),
            pl.BlockSpec((_DIM, _DIM), lambda i: (0, 0)),
            pl.BlockSpec((1, _DIM), lambda i: (0, 0)),
            pl.BlockSpec((_DIM, _DIM), lambda i: (0, 0)),
            pl.BlockSpec((1, _DIM), lambda i: (0, 0)),
        ],
        out_specs=pl.BlockSpec((4, 2, _BLK, _HALF), lambda i: (0, 0, i, 0)),
        out_shape=jax.ShapeDtypeStruct((4, 2, _N, _HALF), jnp.float32),
    )(h_cell, h_net, cnt_src, W1_cell, b1_cell.reshape(1, _DIM),
      W1_net, b1_net.reshape(1, _DIM))


def _out_body(hc_ref, hn_ref, cnt_ref, agg_ref, w4_ref, bsum_ref,
              w2c_ref, b2c_ref, w2n_ref, b2n_ref, yc_ref, yn_ref):
    sin = [lax.rsqrt(jnp.maximum(cnt_ref[:, r], 1.0))[:, None]
           for r in range(4)]

    def conv(r_a, r_b, bias_row):
        acc = bsum_ref[bias_row, :][None, :]
        acc = jnp.broadcast_to(acc, (_BLK, 2 * _DIM)).astype(jnp.float32)
        for r in (r_a, r_b):
            for h in (0, 1):
                acc = acc + jnp.dot(agg_ref[r, h] * sin[r], w4_ref[r, h],
                                    preferred_element_type=jnp.float32)
        return acc

    h2c = _gelu(conv(0, 2, 0))
    h2n = _gelu(conv(1, 3, 1))
    out_c = _gelu(jnp.dot(h2c, w2c_ref[...],
                          preferred_element_type=jnp.float32) + b2c_ref[...])
    out_n = _gelu(jnp.dot(h2n, w2n_ref[...],
                          preferred_element_type=jnp.float32) + b2n_ref[...])
    yc_ref[...] = hc_ref[...] + out_c
    yn_ref[...] = hn_ref[...] + out_n


def _out_call(h_cell, h_net, cnt_dst, agg4, W4, bsum, W2_cell, b2_cell,
              W2_net, b2_net):
    return pl.pallas_call(
        _out_body,
        grid=(_NB,),
        in_specs=[
            pl.BlockSpec((_BLK, _DIM), lambda i: (i, 0)),
            pl.BlockSpec((_BLK, _DIM), lambda i: (i, 0)),
            pl.BlockSpec((_BLK, 4), lambda i: (i, 0)),
            pl.BlockSpec((4, 2, _BLK, _HALF), lambda i: (0, 0, i, 0)),
            pl.BlockSpec((4, 2, _HALF, 2 * _DIM), lambda i: (0, 0, 0, 0)),
            pl.BlockSpec((2, 2 * _DIM), lambda i: (0, 0)),
            pl.BlockSpec((2 * _DIM, _DIM), lambda i: (0, 0)),
            pl.BlockSpec((1, _DIM), lambda i: (0, 0)),
            pl.BlockSpec((2 * _DIM, _DIM), lambda i: (0, 0)),
            pl.BlockSpec((1, _DIM), lambda i: (0, 0)),
        ],
        out_specs=[
            pl.BlockSpec((_BLK, _DIM), lambda i: (i, 0)),
            pl.BlockSpec((_BLK, _DIM), lambda i: (i, 0)),
        ],
        out_shape=[
            jax.ShapeDtypeStruct((_N, _DIM), jnp.float32),
            jax.ShapeDtypeStruct((_N, _DIM), jnp.float32),
        ],
    )(h_cell, h_net, cnt_dst, agg4, W4, bsum,
      W2_cell, b2_cell.reshape(1, _DIM), W2_net, b2_net.reshape(1, _DIM))


_NC = 2
_NS = 16
_CH = 125
_NCHUNK = _E // (_NS * _CH)


def _hist_body(idx_ref, zeros_ref, ones_ref, out_ref, acc, idx_v, ones_v,
               zero_v, sem):
    c = lax.axis_index("c")
    s = lax.axis_index("s")
    pltpu.sync_copy(ones_ref, ones_v)

    pltpu.sync_copy(zeros_ref, zero_v)

    @pl.when(s < 10)
    def _():
        pltpu.sync_copy(zero_v, acc.at[pl.ds(s * 4000, 4000)])

    for k in range(4):
        pltpu.sync_copy(idx_ref.at[4 * c + k, s], idx_v.at[k])
    plsc.subcore_barrier()

    for k in range(4):
        def fire(j, carry, k=k):
            pltpu.async_copy(ones_v, acc.at[idx_v.at[k, j]], sem, add=True)
            return carry
        lax.fori_loop(0, _NCHUNK, fire, 0)

    def drain(j, carry):
        pltpu.make_async_copy(ones_v, acc.at[idx_v.at[0, 0]], sem).wait()
        return carry
    lax.fori_loop(0, 4 * _NCHUNK, drain, 0)
    plsc.subcore_barrier()

    @pl.when(s < 10)
    def _():
        for k in range(4):
            bounce = zero_v.at[pl.ds(0, 1000)]
            pltpu.sync_copy(acc.at[pl.ds(k * _N + s * 1000, 1000)], bounce)
            pltpu.sync_copy(
                bounce,
                out_ref.at[pl.ds((4 * c + k) * _N + s * 1000, 1000)])


def _histograms(idx8):
    mesh = plsc.VectorSubcoreMesh(core_axis_name="c", subcore_axis_name="s")
    hist = functools.partial(
        pl.kernel,
        out_type=jax.ShapeDtypeStruct((8 * _N,), jnp.float32),
        mesh=mesh,
        scratch_types=[
            pltpu.VMEM_SHARED((4 * _N,), jnp.float32),
            pltpu.VMEM((4, _NCHUNK, _CH), jnp.int32),
            pltpu.VMEM((_CH,), jnp.float32),
            pltpu.VMEM((4000,), jnp.float32),
            pltpu.SemaphoreType.DMA,
        ],
    )(_hist_body)
    idx_adj = idx8 + (jnp.arange(8, dtype=jnp.int32) % 4 * _N)[:, None]
    out = hist(idx_adj.reshape(8, _NS, _NCHUNK, _CH),
               jnp.zeros((4000,), jnp.float32),
               jnp.ones((_CH,), jnp.float32))
    return out.reshape(8, _N)


_SCH = 100
_SNC = _E // (_NS * _SCH)
_NQ = 4
_QCH = _SNC // _NQ
_K = 3
_MROWS = [(q * 96, 96) for q in range(10)] + [(960, 40)]


def _seg_body(x_ref, src_ref, dst_ref, zeros_ref, out_ref,
              acc, src_v, dst_v, bufa, bufb, bufc,
              gsa, gsb, gsc, ssa, ssb, ssc):
    c = lax.axis_index("c")
    s = lax.axis_index("s")
    bufs = (bufa, bufb, bufc)
    gsems = (gsa, gsb, gsc)
    ssems = (ssa, ssb, ssc)

    def gwait(i):
        pltpu.make_async_copy(x_ref.at[0, 0].at[src_v.at[0]], bufs[i],
                              gsems[i]).wait()

    def swait(i):
        pltpu.make_async_copy(bufs[i], acc.at[dst_v.at[0]], ssems[i]).wait()

    for r in range(4):
        pltpu.sync_copy(zeros_ref, bufa)

        @pl.when(s < 10)
        def _():
            for off, n in _MROWS:
                pltpu.async_copy(bufa.at[pl.ds(0, n)],
                                 acc.at[pl.ds(s * 1000 + off, n)], ssa)
            for off, n in _MROWS:
                pltpu.make_async_copy(
                    bufa.at[pl.ds(0, n)],
                    acc.at[pl.ds(s * 1000 + off, n)], ssa).wait()
        plsc.subcore_barrier()

        table = x_ref.at[r, c]
        for q4 in range(_NQ):
            pltpu.sync_copy(src_ref.at[r, s, q4], src_v)
            pltpu.sync_copy(dst_ref.at[r, s, q4], dst_v)
            for i in range(_K):
                pltpu.async_copy(table.at[src_v.at[i]], bufs[i], gsems[i])

            def triple(t, carry):
                for i in range(_K):
                    gwait(i)
                    pltpu.async_copy(bufs[i], acc.at[dst_v.at[3 * t + i]],
                                     ssems[i], add=True)
                for i in range(_K):
                    swait(i)

                    @pl.when(3 * t + _K + i < _QCH)
                    def _(i=i):
                        pltpu.async_copy(table.at[src_v.at[3 * t + _K + i]],
                                         bufs[i], gsems[i])
                return carry

            lax.fori_loop(0, _QCH // _K, triple, 0)
            for i in range(_QCH % _K):
                j = (_QCH // _K) * _K + i
                gwait(i)
                pltpu.async_copy(bufs[i], acc.at[dst_v.at[j]],
                                 ssems[i], add=True)
            for i in range(_QCH % _K):
                swait(i)
        plsc.subcore_barrier()

        @pl.when(s < 10)
        def _():
            nm = len(_MROWS)
            for q, (off, n) in enumerate(_MROWS):
                buf = bufs[q % 2].at[pl.ds(0, n)]
                rows = pl.ds(s * 1000 + off, n)
                if q >= 2:
                    poff, pn = _MROWS[q - 2]
                    pbuf = bufs[q % 2].at[pl.ds(0, pn)]
                    prows = pl.ds(s * 1000 + poff, pn)
                    pltpu.make_async_copy(pbuf, out_ref.at[r, c].at[prows],
                                          gsa).wait()
                pltpu.async_copy(acc.at[rows], buf, ssa)
                pltpu.make_async_copy(acc.at[rows], buf, ssa).wait()
                pltpu.async_copy(buf, out_ref.at[r, c].at[rows], gsa)
            for q in (nm - 2, nm - 1):
                off, n = _MROWS[q]
                buf = bufs[q % 2].at[pl.ds(0, n)]
                rows = pl.ds(s * 1000 + off, n)
                pltpu.make_async_copy(buf, out_ref.at[r, c].at[rows],
                                      gsa).wait()
        plsc.subcore_barrier()


def _segsums(x4, edges):
    mesh = plsc.VectorSubcoreMesh(core_axis_name="c", subcore_axis_name="s")
    seg = functools.partial(
        pl.kernel,
        out_type=jax.ShapeDtypeStruct((4, 2, _N, _HALF), jnp.float32),
        mesh=mesh,
        scratch_types=[
            pltpu.VMEM_SHARED((_N, _HALF), jnp.float32),
            pltpu.VMEM((_QCH, _SCH), jnp.int32),
            pltpu.VMEM((_QCH, _SCH), jnp.int32),
            pltpu.VMEM((_SCH, _HALF), jnp.float32),
            pltpu.VMEM((_SCH, _HALF), jnp.float32),
            pltpu.VMEM((_SCH, _HALF), jnp.float32),
            pltpu.SemaphoreType.DMA,
            pltpu.SemaphoreType.DMA,
            pltpu.SemaphoreType.DMA,
            pltpu.SemaphoreType.DMA,
            pltpu.SemaphoreType.DMA,
            pltpu.SemaphoreType.DMA,
        ],
    )(_seg_body)
    src4 = jnp.stack([e[0] for e in edges]).reshape(4, _NS, _NQ, _QCH, _SCH)
    dst4 = jnp.stack([e[1] for e in edges]).reshape(4, _NS, _NQ, _QCH, _SCH)
    return seg(x4, src4, dst4, jnp.zeros((_SCH, _HALF), jnp.float32))


def kernel(h_cell, h_net, edge_cc, edge_cn, edge_nc, edge_nn,
           W1_cell, b1_cell, W1_net, b1_net,
           Wcc, bcc, Wcn, bcn, Wnc, bnc, Wnn, bnn,
           W2_cell, b2_cell, W2_net, b2_net):
    edges = [edge_cc, edge_cn, edge_nc, edge_nn]
    idx8 = jnp.stack([e[0] for e in edges] + [e[1] for e in edges])

    counts = _histograms(idx8)
    cnt_src = counts[:4].T
    cnt_dst = counts[4:].T

    x4 = _lin1_call(h_cell, h_net, cnt_src, W1_cell, b1_cell, W1_net, b1_net)

    agg4 = _segsums(x4, edges)

    W4 = jnp.stack([jnp.stack([W[:_HALF], W[_HALF:]])
                    for W in (Wcc, Wcn, Wnc, Wnn)])
    bsum = jnp.stack([bcc + bnc, bcn + bnn])

    yc, yn = _out_call(h_cell, h_net, cnt_dst, agg4, W4, bsum,
                       W2_cell, b2_cell, W2_net, b2_net)
    return (yc, yn)


# speedup vs baseline: 1.1369x; 1.0388x over previous
import functools

import jax
import jax.numpy as jnp
from jax import lax
from jax.experimental import pallas as pl
from jax.experimental.pallas import tpu as pltpu
from jax.experimental.pallas import tpu_sc as plsc

_DIM = 256
_N = 10000
_E = 160000
_HALF = 128
_NB = 10
_BLK = _N // _NB


def _gelu(x):
    return 0.5 * x * (1.0 + lax.erf(x * 0.7071067811865476))


def _lin1_body(hc_ref, hn_ref, cnt_ref, w1c_ref, b1c_ref, w1n_ref, b1n_ref,
               x4_ref):
    h1c = _gelu(jnp.dot(hc_ref[...], w1c_ref[...],
                        preferred_element_type=jnp.float32) + b1c_ref[...])
    h1n = _gelu(jnp.dot(hn_ref[...], w1n_ref[...],
                        preferred_element_type=jnp.float32) + b1n_ref[...])
    for r, h1 in ((0, h1c), (1, h1c), (2, h1n), (3, h1n)):
        s = lax.rsqrt(jnp.maximum(cnt_ref[:, r], 1.0))[:, None]
        xs = h1 * s
        x4_ref[r, 0] = xs[:, :_HALF]
        x4_ref[r, 1] = xs[:, _HALF:]


def _lin1_call(h_cell, h_net, cnt_src, W1_cell, b1_cell, W1_net, b1_net):
    return pl.pallas_call(
        _lin1_body,
        grid=(_NB,),
        in_specs=[
            pl.BlockSpec((_BLK, _DIM), lambda i: (i, 0)),
            pl.BlockSpec((_BLK, _DIM), lambda i: (i, 0)),
            pl.BlockSpec((_BLK, 4), lambda i: (i, 0)),
            pl.BlockSpec((_DIM, _DIM), lambda i: (0, 0)),
            pl.BlockSpec((1, _DIM), lambda i: (0, 0)),
            pl.BlockSpec((_DIM, _DIM), lambda i: (0, 0)),
            pl.BlockSpec((1, _DIM), lambda i: (0, 0)),
        ],
        out_specs=pl.BlockSpec((4, 2, _BLK, _HALF), lambda i: (0, 0, i, 0)),
        out_shape=jax.ShapeDtypeStruct((4, 2, _N, _HALF), jnp.float32),
    )(h_cell, h_net, cnt_src, W1_cell, b1_cell.reshape(1, _DIM),
      W1_net, b1_net.reshape(1, _DIM))


def _out_body(hc_ref, hn_ref, cnt_ref, agg_ref, w4_ref, bsum_ref,
              w2c_ref, b2c_ref, w2n_ref, b2n_ref, yc_ref, yn_ref):
    sin = [lax.rsqrt(jnp.maximum(cnt_ref[:, r], 1.0))[:, None]
           for r in range(4)]

    def conv(r_a, r_b, bias_row):
        acc = bsum_ref[bias_row, :][None, :]
        acc = jnp.broadcast_to(acc, (_BLK, 2 * _DIM)).astype(jnp.float32)
        for r in (r_a, r_b):
            for h in (0, 1):
                acc = acc + jnp.dot(agg_ref[r, h] * sin[r], w4_ref[r, h],
                                    preferred_element_type=jnp.float32)
        return acc

    h2c = _gelu(conv(0, 2, 0))
    h2n = _gelu(conv(1, 3, 1))
    out_c = _gelu(jnp.dot(h2c, w2c_ref[...],
                          preferred_element_type=jnp.float32) + b2c_ref[...])
    out_n = _gelu(jnp.dot(h2n, w2n_ref[...],
                          preferred_element_type=jnp.float32) + b2n_ref[...])
    yc_ref[...] = hc_ref[...] + out_c
    yn_ref[...] = hn_ref[...] + out_n


def _out_call(h_cell, h_net, cnt_dst, agg4, W4, bsum, W2_cell, b2_cell,
              W2_net, b2_net):
    return pl.pallas_call(
        _out_body,
        grid=(_NB,),
        in_specs=[
            pl.BlockSpec((_BLK, _DIM), lambda i: (i, 0)),
            pl.BlockSpec((_BLK, _DIM), lambda i: (i, 0)),
            pl.BlockSpec((_BLK, 4), lambda i: (i, 0)),
            pl.BlockSpec((4, 2, _BLK, _HALF), lambda i: (0, 0, i, 0)),
            pl.BlockSpec((4, 2, _HALF, 2 * _DIM), lambda i: (0, 0, 0, 0)),
            pl.BlockSpec((2, 2 * _DIM), lambda i: (0, 0)),
            pl.BlockSpec((2 * _DIM, _DIM), lambda i: (0, 0)),
            pl.BlockSpec((1, _DIM), lambda i: (0, 0)),
            pl.BlockSpec((2 * _DIM, _DIM), lambda i: (0, 0)),
            pl.BlockSpec((1, _DIM), lambda i: (0, 0)),
        ],
        out_specs=[
            pl.BlockSpec((_BLK, _DIM), lambda i: (i, 0)),
            pl.BlockSpec((_BLK, _DIM), lambda i: (i, 0)),
        ],
        out_shape=[
            jax.ShapeDtypeStruct((_N, _DIM), jnp.float32),
            jax.ShapeDtypeStruct((_N, _DIM), jnp.float32),
        ],
    )(h_cell, h_net, cnt_dst, agg4, W4, bsum,
      W2_cell, b2_cell.reshape(1, _DIM), W2_net, b2_net.reshape(1, _DIM))


_NC = 2
_NS = 16
_CH = 125
_NCHUNK = _E // (_NS * _CH)


def _hist_body(idx_ref, zeros_ref, ones_ref, out_ref, acc, idx_v, ones_v,
               zero_v, sem):
    c = lax.axis_index("c")
    s = lax.axis_index("s")
    pltpu.sync_copy(ones_ref, ones_v)

    pltpu.sync_copy(zeros_ref, zero_v)

    @pl.when(s < 10)
    def _():
        pltpu.sync_copy(zero_v, acc.at[pl.ds(s * 4000, 4000)])

    for k in range(4):
        pltpu.sync_copy(idx_ref.at[4 * c + k, s], idx_v.at[k])
    plsc.subcore_barrier()

    for k in range(4):
        def fire(j, carry, k=k):
            pltpu.async_copy(ones_v, acc.at[idx_v.at[k, j]], sem, add=True)
            return carry
        lax.fori_loop(0, _NCHUNK, fire, 0)

    def drain(j, carry):
        pltpu.make_async_copy(ones_v, acc.at[idx_v.at[0, 0]], sem).wait()
        return carry
    lax.fori_loop(0, 4 * _NCHUNK, drain, 0)
    plsc.subcore_barrier()

    @pl.when(s < 10)
    def _():
        for k in range(4):
            bounce = zero_v.at[pl.ds(0, 1000)]
            pltpu.sync_copy(acc.at[pl.ds(k * _N + s * 1000, 1000)], bounce)
            pltpu.sync_copy(
                bounce,
                out_ref.at[pl.ds((4 * c + k) * _N + s * 1000, 1000)])


def _histograms(idx8):
    mesh = plsc.VectorSubcoreMesh(core_axis_name="c", subcore_axis_name="s")
    hist = functools.partial(
        pl.kernel,
        out_type=jax.ShapeDtypeStruct((8 * _N,), jnp.float32),
        mesh=mesh,
        scratch_types=[
            pltpu.VMEM_SHARED((4 * _N,), jnp.float32),
            pltpu.VMEM((4, _NCHUNK, _CH), jnp.int32),
            pltpu.VMEM((_CH,), jnp.float32),
            pltpu.VMEM((4000,), jnp.float32),
            pltpu.SemaphoreType.DMA,
        ],
    )(_hist_body)
    idx_adj = idx8 + (jnp.arange(8, dtype=jnp.int32) % 4 * _N)[:, None]
    out = hist(idx_adj.reshape(8, _NS, _NCHUNK, _CH),
               jnp.zeros((4000,), jnp.float32),
               jnp.ones((_CH,), jnp.float32))
    return out.reshape(8, _N)


_SCH = 80
_SNC = _E // (_NS * _SCH)
_NQ = 5
_QCH = _SNC // _NQ
_K = 4
_MROWS = [(q * 80, 80) for q in range(12)] + [(960, 40)]


def _seg_body(x_ref, src_ref, dst_ref, zeros_ref, out_ref,
              acc, src_v, dst_v, bufa, bufb, bufc, bufd,
              gsa, gsb, gsc, gsd, ssa, ssb, ssc, ssd):
    c = lax.axis_index("c")
    s = lax.axis_index("s")
    bufs = (bufa, bufb, bufc, bufd)
    gsems = (gsa, gsb, gsc, gsd)
    ssems = (ssa, ssb, ssc, ssd)

    def gwait(i):
        pltpu.make_async_copy(x_ref.at[0, 0].at[src_v.at[0]], bufs[i],
                              gsems[i]).wait()

    def swait(i):
        pltpu.make_async_copy(bufs[i], acc.at[dst_v.at[0]], ssems[i]).wait()

    for r in range(4):
        pltpu.sync_copy(zeros_ref, bufa)

        @pl.when(s < 10)
        def _():
            for off, n in _MROWS:
                pltpu.async_copy(bufa.at[pl.ds(0, n)],
                                 acc.at[pl.ds(s * 1000 + off, n)], ssa)
            for off, n in _MROWS:
                pltpu.make_async_copy(
                    bufa.at[pl.ds(0, n)],
                    acc.at[pl.ds(s * 1000 + off, n)], ssa).wait()
        plsc.subcore_barrier()

        table = x_ref.at[r, c]
        for q4 in range(_NQ):
            pltpu.sync_copy(src_ref.at[r, s, q4], src_v)
            pltpu.sync_copy(dst_ref.at[r, s, q4], dst_v)
            for i in range(_K):
                pltpu.async_copy(table.at[src_v.at[i]], bufs[i], gsems[i])

            def kgroup(t, carry):
                for i in range(_K):
                    gwait(i)
                    pltpu.async_copy(bufs[i], acc.at[dst_v.at[_K * t + i]],
                                     ssems[i], add=True)
                for i in range(_K):
                    swait(i)

                    @pl.when(_K * t + _K + i < _QCH)
                    def _(i=i):
                        pltpu.async_copy(
                            table.at[src_v.at[_K * t + _K + i]],
                            bufs[i], gsems[i])
                return carry

            lax.fori_loop(0, _QCH // _K, kgroup, 0)
            for i in range(_QCH % _K):
                j = (_QCH // _K) * _K + i
                gwait(i)
                pltpu.async_copy(bufs[i], acc.at[dst_v.at[j]],
                                 ssems[i], add=True)
            for i in range(_QCH % _K):
                swait(i)
        plsc.subcore_barrier()

        @pl.when(s < 10)
        def _():
            nm = len(_MROWS)
            for q, (off, n) in enumerate(_MROWS):
                buf = bufs[q % 2].at[pl.ds(0, n)]
                rows = pl.ds(s * 1000 + off, n)
                if q >= 2:
                    poff, pn = _MROWS[q - 2]
                    pbuf = bufs[q % 2].at[pl.ds(0, pn)]
                    prows = pl.ds(s * 1000 + poff, pn)
                    pltpu.make_async_copy(pbuf, out_ref.at[r, c].at[prows],
                                          gsa).wait()
                pltpu.async_copy(acc.at[rows], buf, ssa)
                pltpu.make_async_copy(acc.at[rows], buf, ssa).wait()
                pltpu.async_copy(buf, out_ref.at[r, c].at[rows], gsa)
            for q in (nm - 2, nm - 1):
                off, n = _MROWS[q]
                buf = bufs[q % 2].at[pl.ds(0, n)]
                rows = pl.ds(s * 1000 + off, n)
                pltpu.make_async_copy(buf, out_ref.at[r, c].at[rows],
                                      gsa).wait()
        plsc.subcore_barrier()


def _segsums(x4, edges):
    mesh = plsc.VectorSubcoreMesh(core_axis_name="c", subcore_axis_name="s")
    seg = functools.partial(
        pl.kernel,
        out_type=jax.ShapeDtypeStruct((4, 2, _N, _HALF), jnp.float32),
        mesh=mesh,
        scratch_types=[
            pltpu.VMEM_SHARED((_N, _HALF), jnp.float32),
            pltpu.VMEM((_QCH, _SCH), jnp.int32),
            pltpu.VMEM((_QCH, _SCH), jnp.int32),
            pltpu.VMEM((_SCH, _HALF), jnp.float32),
            pltpu.VMEM((_SCH, _HALF), jnp.float32),
            pltpu.VMEM((_SCH, _HALF), jnp.float32),
            pltpu.VMEM((_SCH, _HALF), jnp.float32),
            pltpu.SemaphoreType.DMA,
            pltpu.SemaphoreType.DMA,
            pltpu.SemaphoreType.DMA,
            pltpu.SemaphoreType.DMA,
            pltpu.SemaphoreType.DMA,
            pltpu.SemaphoreType.DMA,
            pltpu.SemaphoreType.DMA,
            pltpu.SemaphoreType.DMA,
        ],
    )(_seg_body)
    src4 = jnp.stack([e[0] for e in edges]).reshape(4, _NS, _NQ, _QCH, _SCH)
    dst4 = jnp.stack([e[1] for e in edges]).reshape(4, _NS, _NQ, _QCH, _SCH)
    return seg(x4, src4, dst4, jnp.zeros((_SCH, _HALF), jnp.float32))


def kernel(h_cell, h_net, edge_cc, edge_cn, edge_nc, edge_nn,
           W1_cell, b1_cell, W1_net, b1_net,
           Wcc, bcc, Wcn, bcn, Wnc, bnc, Wnn, bnn,
           W2_cell, b2_cell, W2_net, b2_net):
    edges = [edge_cc, edge_cn, edge_nc, edge_nn]
    idx8 = jnp.stack([e[0] for e in edges] + [e[1] for e in edges])

    counts = _histograms(idx8)
    cnt_src = counts[:4].T
    cnt_dst = counts[4:].T

    x4 = _lin1_call(h_cell, h_net, cnt_src, W1_cell, b1_cell, W1_net, b1_net)

    agg4 = _segsums(x4, edges)

    W4 = jnp.stack([jnp.stack([W[:_HALF], W[_HALF:]])
                    for W in (Wcc, Wcn, Wnc, Wnn)])
    bsum = jnp.stack([bcc + bnc, bcn + bnn])

    yc, yn = _out_call(h_cell, h_net, cnt_dst, agg4, W4, bsum,
                       W2_cell, b2_cell, W2_net, b2_net)
    return (yc, yn)
